# baseline (speedup 1.0000x reference)
import jax
import jax.numpy as jnp
from jax import lax
from jax.experimental import pallas as pl
from jax.experimental.pallas import tpu as pltpu
from jax.experimental.pallas import tpu_sc as plsc

N = 10000
E = 160000
F_IN = 128
F_E = 16
H = 32

NC = 2
NS = 16
NW = NC * NS
CHUNK = 128
GROUP = 10
N_PAD = 10240
E_PAD = NW * 40 * CHUNK
CPT = E_PAD // NW // CHUNK
ROWS_PT = N_PAD // NS
DEGW = 16

_mesh = plsc.VectorSubcoreMesh(core_axis_name="c", subcore_axis_name="s")
_sc_params = pltpu.CompilerParams(use_tc_tiling_on_sc=False)


def _gather_body(h_hbm, src2_hbm, out_hbm, idx_v, rows_v, sem):
    cid = lax.axis_index("c")
    sid = lax.axis_index("s")
    wid = sid * NC + cid
    pltpu.sync_copy(src2_hbm.at[pl.ds(wid * CPT, CPT)], idx_v)
    for g in range(CPT // GROUP):
        descs = [
            pltpu.async_copy(
                h_hbm.at[idx_v.at[g * GROUP + j]],
                rows_v.at[pl.ds(j * CHUNK, CHUNK)],
                sem,
            )
            for j in range(GROUP)
        ]
        for d in descs:
            d.wait()
        base = pl.multiple_of((wid * CPT + g * GROUP) * CHUNK, CHUNK)
        pltpu.sync_copy(rows_v, out_hbm.at[pl.ds(base, GROUP * CHUNK)])


_gather = pl.kernel(
    _gather_body,
    out_type=jax.ShapeDtypeStruct((E_PAD, H), jnp.float32),
    mesh=_mesh,
    scratch_types=[
        pltpu.VMEM((CPT, CHUNK), jnp.int32),
        pltpu.VMEM((GROUP * CHUNK, H), jnp.float32),
        pltpu.SemaphoreType.DMA,
    ],
    compiler_params=_sc_params,
)


def _make_scatter(with_deg):
    def body(*refs):
        if with_deg:
            (m_hbm, dst2_hbm, zeros_hbm, zdeg_hbm, onecol_hbm,
             s_out, deg_out, idx_v, val_v, ones_v, sh_s, sh_deg) = refs
        else:
            (m_hbm, dst2_hbm, zeros_hbm,
             s_out, idx_v, val_v, sh_s) = refs
        cid = lax.axis_index("c")
        sid = lax.axis_index("s")
        wid = sid * NC + cid
        row0 = pl.multiple_of(sid * ROWS_PT, ROWS_PT)
        pltpu.sync_copy(zeros_hbm, sh_s.at[pl.ds(row0, ROWS_PT)])
        if with_deg:
            pltpu.sync_copy(zdeg_hbm, sh_deg.at[pl.ds(row0, ROWS_PT)])
            pltpu.sync_copy(onecol_hbm, ones_v)
        pltpu.sync_copy(dst2_hbm.at[pl.ds(wid * CPT, CPT)], idx_v)
        plsc.subcore_barrier()
        for g in range(CPT // GROUP):
            base = pl.multiple_of((wid * CPT + g * GROUP) * CHUNK, CHUNK)
            pltpu.sync_copy(m_hbm.at[pl.ds(base, GROUP * CHUNK)], val_v)
            for j in range(GROUP):
                idx_row = idx_v.at[g * GROUP + j]
                pltpu.sync_copy(
                    val_v.at[pl.ds(j * CHUNK, CHUNK)],
                    sh_s.at[idx_row],
                    add=True,
                )
                if with_deg:
                    pltpu.sync_copy(ones_v, sh_deg.at[idx_row], add=True)
        plsc.subcore_barrier()
        obase = pl.multiple_of(cid * N_PAD + row0, ROWS_PT)
        pltpu.sync_copy(sh_s.at[pl.ds(row0, ROWS_PT)],
                        s_out.at[pl.ds(obase, ROWS_PT)])
        if with_deg:
            pltpu.sync_copy(sh_deg.at[pl.ds(row0, ROWS_PT)],
                            deg_out.at[pl.ds(obase, ROWS_PT)])

    out_type = [jax.ShapeDtypeStruct((NC * N_PAD, H), jnp.float32)]
    scratch = [
        pltpu.VMEM((CPT, CHUNK), jnp.int32),
        pltpu.VMEM((GROUP * CHUNK, H), jnp.float32),
    ]
    if with_deg:
        out_type.append(jax.ShapeDtypeStruct((NC * N_PAD, DEGW), jnp.float32))
        scratch.append(pltpu.VMEM((CHUNK, DEGW), jnp.float32))
    scratch.append(pltpu.VMEM_SHARED((N_PAD, H), jnp.float32))
    if with_deg:
        scratch.append(pltpu.VMEM_SHARED((N_PAD, DEGW), jnp.float32))
    return pl.kernel(
        body,
        out_type=tuple(out_type) if with_deg else out_type[0],
        mesh=_mesh,
        scratch_types=scratch,
        compiler_params=_sc_params,
    )


_scatter_deg = _make_scatter(True)
_scatter = _make_scatter(False)


def _proj_body(nf_ref, wp_ref, bp_ref, out_ref):
    out_ref[...] = (
        jnp.dot(nf_ref[...], wp_ref[...], preferred_element_type=jnp.float32)
        + bp_ref[...]
    )


_NB = 1000

_proj = pl.pallas_call(
    _proj_body,
    grid=(N // _NB,),
    in_specs=[
        pl.BlockSpec((_NB, F_IN), lambda i: (i, 0)),
        pl.BlockSpec((F_IN, H), lambda i: (0, 0)),
        pl.BlockSpec((1, H), lambda i: (0, 0)),
    ],
    out_specs=pl.BlockSpec((_NB, H), lambda i: (i, 0)),
    out_shape=jax.ShapeDtypeStruct((N, H), jnp.float32),
)


_BE = 640


def _msg_body(ef_ref, hs_ref, we_ref, be_ref, out_ref):
    z = (
        jnp.dot(ef_ref[...], we_ref[...], preferred_element_type=jnp.float32)
        + be_ref[...]
    )
    z = jnp.maximum(z, 0.0).reshape(_BE, H, H)
    out_ref[...] = jnp.sum(z * hs_ref[...][:, :, None], axis=1)


_msg = pl.pallas_call(
    _msg_body,
    grid=(E_PAD // _BE,),
    in_specs=[
        pl.BlockSpec((_BE, F_E), lambda i: (i, 0)),
        pl.BlockSpec((_BE, H), lambda i: (i, 0)),
        pl.BlockSpec((F_E, H * H), lambda i: (0, 0)),
        pl.BlockSpec((1, H * H), lambda i: (0, 0)),
    ],
    out_specs=pl.BlockSpec((_BE, H), lambda i: (i, 0)),
    out_shape=jax.ShapeDtypeStruct((E_PAD, H), jnp.float32),
)


def _combine_body(sp_ref, dp_ref, b_ref, out_ref):
    s = sp_ref[0] + sp_ref[1]
    deg = jnp.maximum(dp_ref[0, :, 0:1] + dp_ref[1, :, 0:1], 1.0)
    out_ref[...] = jnp.maximum(s / deg + b_ref[...], 0.0)


_combine = pl.pallas_call(
    _combine_body,
    grid=(N // _NB,),
    in_specs=[
        pl.BlockSpec((NC, _NB, H), lambda i: (0, i, 0)),
        pl.BlockSpec((NC, _NB, DEGW), lambda i: (0, i, 0)),
        pl.BlockSpec((1, H), lambda i: (0, 0)),
    ],
    out_specs=pl.BlockSpec((_NB, H), lambda i: (i, 0)),
    out_shape=jax.ShapeDtypeStruct((N, H), jnp.float32),
)


def kernel(node_features, edge_index, edge_features, W_proj, b_proj,
           W_e, b_e, bias1, bias2):
    src = edge_index[0]
    dst = edge_index[1]
    src2 = jnp.concatenate(
        [src, jnp.zeros((E_PAD - E,), jnp.int32)]
    ).reshape(E_PAD // CHUNK, CHUNK)
    dst2 = jnp.concatenate(
        [dst, jnp.full((E_PAD - E,), N_PAD - 1, jnp.int32)]
    ).reshape(E_PAD // CHUNK, CHUNK)
    ef_p = jnp.concatenate(
        [edge_features, jnp.zeros((E_PAD - E, F_E), jnp.float32)], axis=0
    )
    zeros = jnp.zeros((ROWS_PT, H), jnp.float32)
    zdeg = jnp.zeros((ROWS_PT, DEGW), jnp.float32)
    onecol = jnp.zeros((CHUNK, DEGW), jnp.float32).at[:, 0].set(1.0)
    be2 = b_e.reshape(1, H * H)

    h = _proj(node_features, W_proj, b_proj.reshape(1, H))

    hs = _gather(h, src2)
    m = _msg(ef_p, hs, W_e, be2)
    s_flat, deg_flat = _scatter_deg(m, dst2, zeros, zdeg, onecol)
    sp = s_flat.reshape(NC, N_PAD, H)
    dp = deg_flat.reshape(NC, N_PAD, DEGW)
    h = _combine(sp, dp, bias1.reshape(1, H))

    hs = _gather(h, src2)
    m = _msg(ef_p, hs, W_e, be2)
    s_flat = _scatter(m, dst2, zeros)
    sp = s_flat.reshape(NC, N_PAD, H)
    h = _combine(sp, dp, bias2.reshape(1, H))
    return h


# speedup vs baseline: 3.1389x; 3.1389x over previous
import jax
import jax.numpy as jnp
from jax import lax
from jax.experimental import pallas as pl
from jax.experimental.pallas import tpu as pltpu
from jax.experimental.pallas import tpu_sc as plsc

N = 10000
E = 160000
F_IN = 128
F_E = 16
H = 32

NC = 2
NS = 16
NW = NC * NS
CHUNK = 128
GROUP = 10
N_PAD = 10240
E_PAD = NW * 40 * CHUNK
CPT = E_PAD // NW // CHUNK
ROWS_PT = N_PAD // NS
DEGW = 16

_mesh = plsc.VectorSubcoreMesh(core_axis_name="c", subcore_axis_name="s")
_sc_params = pltpu.CompilerParams(use_tc_tiling_on_sc=False)


def _gather_body(h_hbm, src2_hbm, out_hbm, idx_v, rows_v, sem):
    cid = lax.axis_index("c")
    sid = lax.axis_index("s")
    wid = sid * NC + cid
    pltpu.sync_copy(src2_hbm.at[pl.ds(wid * CPT, CPT)], idx_v)
    for g in range(CPT // GROUP):
        descs = [
            pltpu.async_copy(
                h_hbm.at[idx_v.at[g * GROUP + j]],
                rows_v.at[pl.ds(j * CHUNK, CHUNK)],
                sem,
            )
            for j in range(GROUP)
        ]
        for d in descs:
            d.wait()
        base = pl.multiple_of((wid * CPT + g * GROUP) * CHUNK, CHUNK)
        pltpu.sync_copy(rows_v, out_hbm.at[pl.ds(base, GROUP * CHUNK)])


_gather = pl.kernel(
    _gather_body,
    out_type=jax.ShapeDtypeStruct((E_PAD, H), jnp.float32),
    mesh=_mesh,
    scratch_types=[
        pltpu.VMEM((CPT, CHUNK), jnp.int32),
        pltpu.VMEM((GROUP * CHUNK, H), jnp.float32),
        pltpu.SemaphoreType.DMA,
    ],
    compiler_params=_sc_params,
)


def _make_scatter(with_deg):
    def body(*refs):
        if with_deg:
            (m_hbm, dst2_hbm, zeros_hbm, zdeg_hbm, onecol_hbm,
             s_out, deg_out, idx_v, val_v, ones_v, sh_s, sh_deg) = refs
        else:
            (m_hbm, dst2_hbm, zeros_hbm,
             s_out, idx_v, val_v, sh_s) = refs
        cid = lax.axis_index("c")
        sid = lax.axis_index("s")
        wid = sid * NC + cid
        row0 = pl.multiple_of(sid * ROWS_PT, ROWS_PT)
        pltpu.sync_copy(zeros_hbm, sh_s.at[pl.ds(row0, ROWS_PT)])
        if with_deg:
            pltpu.sync_copy(zdeg_hbm, sh_deg.at[pl.ds(row0, ROWS_PT)])
            pltpu.sync_copy(onecol_hbm, ones_v)
        pltpu.sync_copy(dst2_hbm.at[pl.ds(wid * CPT, CPT)], idx_v)
        plsc.subcore_barrier()
        for g in range(CPT // GROUP):
            base = pl.multiple_of((wid * CPT + g * GROUP) * CHUNK, CHUNK)
            pltpu.sync_copy(m_hbm.at[pl.ds(base, GROUP * CHUNK)], val_v)
            for j in range(GROUP):
                idx_row = idx_v.at[g * GROUP + j]
                pltpu.sync_copy(
                    val_v.at[pl.ds(j * CHUNK, CHUNK)],
                    sh_s.at[idx_row],
                    add=True,
                )
                if with_deg:
                    pltpu.sync_copy(ones_v, sh_deg.at[idx_row], add=True)
        plsc.subcore_barrier()
        obase = pl.multiple_of(cid * N_PAD + row0, ROWS_PT)
        pltpu.sync_copy(sh_s.at[pl.ds(row0, ROWS_PT)],
                        s_out.at[pl.ds(obase, ROWS_PT)])
        if with_deg:
            pltpu.sync_copy(sh_deg.at[pl.ds(row0, ROWS_PT)],
                            deg_out.at[pl.ds(obase, ROWS_PT)])

    out_type = [jax.ShapeDtypeStruct((NC * N_PAD, H), jnp.float32)]
    scratch = [
        pltpu.VMEM((CPT, CHUNK), jnp.int32),
        pltpu.VMEM((GROUP * CHUNK, H), jnp.float32),
    ]
    if with_deg:
        out_type.append(jax.ShapeDtypeStruct((NC * N_PAD, DEGW), jnp.float32))
        scratch.append(pltpu.VMEM((CHUNK, DEGW), jnp.float32))
    scratch.append(pltpu.VMEM_SHARED((N_PAD, H), jnp.float32))
    if with_deg:
        scratch.append(pltpu.VMEM_SHARED((N_PAD, DEGW), jnp.float32))
    return pl.kernel(
        body,
        out_type=tuple(out_type) if with_deg else out_type[0],
        mesh=_mesh,
        scratch_types=scratch,
        compiler_params=_sc_params,
    )


_scatter_deg = _make_scatter(True)
_scatter = _make_scatter(False)


def _proj_body(nf_ref, wp_ref, bp_ref, out_ref):
    out_ref[...] = (
        jnp.dot(nf_ref[...], wp_ref[...], preferred_element_type=jnp.float32)
        + bp_ref[...]
    )


_NB = 1000

_proj = pl.pallas_call(
    _proj_body,
    grid=(N // _NB,),
    in_specs=[
        pl.BlockSpec((_NB, F_IN), lambda i: (i, 0)),
        pl.BlockSpec((F_IN, H), lambda i: (0, 0)),
        pl.BlockSpec((1, H), lambda i: (0, 0)),
    ],
    out_specs=pl.BlockSpec((_NB, H), lambda i: (i, 0)),
    out_shape=jax.ShapeDtypeStruct((N, H), jnp.float32),
)


_BE = 640


def _msg_body(efT_ref, hs_ref, weT_ref, be_ref, out_ref):
    z = (
        jnp.dot(weT_ref[...], efT_ref[...], preferred_element_type=jnp.float32)
        + be_ref[...]
    )
    z = jnp.maximum(z, 0.0)
    hsT = hs_ref[...].T
    acc = z[0:H, :] * hsT[0:1, :]
    for i in range(1, H):
        acc = acc + z[H * i:H * i + H, :] * hsT[i:i + 1, :]
    out_ref[...] = acc.T


_msg = pl.pallas_call(
    _msg_body,
    grid=(E_PAD // _BE,),
    in_specs=[
        pl.BlockSpec((F_E, _BE), lambda i: (0, i)),
        pl.BlockSpec((_BE, H), lambda i: (i, 0)),
        pl.BlockSpec((H * H, F_E), lambda i: (0, 0)),
        pl.BlockSpec((H * H, 1), lambda i: (0, 0)),
    ],
    out_specs=pl.BlockSpec((_BE, H), lambda i: (i, 0)),
    out_shape=jax.ShapeDtypeStruct((E_PAD, H), jnp.float32),
)


def _combine_body(sp_ref, dp_ref, b_ref, out_ref):
    s = sp_ref[0] + sp_ref[1]
    deg = jnp.maximum(dp_ref[0, :, 0:1] + dp_ref[1, :, 0:1], 1.0)
    out_ref[...] = jnp.maximum(s / deg + b_ref[...], 0.0)


_combine = pl.pallas_call(
    _combine_body,
    grid=(N // _NB,),
    in_specs=[
        pl.BlockSpec((NC, _NB, H), lambda i: (0, i, 0)),
        pl.BlockSpec((NC, _NB, DEGW), lambda i: (0, i, 0)),
        pl.BlockSpec((1, H), lambda i: (0, 0)),
    ],
    out_specs=pl.BlockSpec((_NB, H), lambda i: (i, 0)),
    out_shape=jax.ShapeDtypeStruct((N, H), jnp.float32),
)


def kernel(node_features, edge_index, edge_features, W_proj, b_proj,
           W_e, b_e, bias1, bias2):
    src = edge_index[0]
    dst = edge_index[1]
    src2 = jnp.concatenate(
        [src, jnp.zeros((E_PAD - E,), jnp.int32)]
    ).reshape(E_PAD // CHUNK, CHUNK)
    dst2 = jnp.concatenate(
        [dst, jnp.full((E_PAD - E,), N_PAD - 1, jnp.int32)]
    ).reshape(E_PAD // CHUNK, CHUNK)
    efT = jnp.concatenate(
        [edge_features, jnp.zeros((E_PAD - E, F_E), jnp.float32)], axis=0
    ).T
    weT = W_e.T
    zeros = jnp.zeros((ROWS_PT, H), jnp.float32)
    zdeg = jnp.zeros((ROWS_PT, DEGW), jnp.float32)
    onecol = jnp.zeros((CHUNK, DEGW), jnp.float32).at[:, 0].set(1.0)
    be2 = b_e.reshape(H * H, 1)

    h = _proj(node_features, W_proj, b_proj.reshape(1, H))

    hs = _gather(h, src2)
    m = _msg(efT, hs, weT, be2)
    s_flat, deg_flat = _scatter_deg(m, dst2, zeros, zdeg, onecol)
    sp = s_flat.reshape(NC, N_PAD, H)
    dp = deg_flat.reshape(NC, N_PAD, DEGW)
    h = _combine(sp, dp, bias1.reshape(1, H))

    hs = _gather(h, src2)
    m = _msg(efT, hs, weT, be2)
    s_flat = _scatter(m, dst2, zeros)
    sp = s_flat.reshape(NC, N_PAD, H)
    h = _combine(sp, dp, bias2.reshape(1, H))
    return h


# speedup vs baseline: 3.6582x; 1.1654x over previous
import jax
import jax.numpy as jnp
from jax import lax
from jax.experimental import pallas as pl
from jax.experimental.pallas import tpu as pltpu
from jax.experimental.pallas import tpu_sc as plsc

N = 10000
E = 160000
F_IN = 128
F_E = 16
H = 32

NC = 2
NS = 16
NW = NC * NS
CHUNK = 128
GROUP = 10
N_PAD = 10240
E_PAD = NW * 40 * CHUNK
CPT = E_PAD // NW // CHUNK
ROWS_PT = N_PAD // NS
DEGW = 16

_mesh = plsc.VectorSubcoreMesh(core_axis_name="c", subcore_axis_name="s")
_sc_params = pltpu.CompilerParams(use_tc_tiling_on_sc=False)


def _gather_body(h_hbm, src2_hbm, out_hbm, idx_v, rows_v, sem):
    cid = lax.axis_index("c")
    sid = lax.axis_index("s")
    wid = sid * NC + cid
    pltpu.sync_copy(src2_hbm.at[pl.ds(wid * CPT, CPT)], idx_v)
    for g in range(CPT // GROUP):
        descs = [
            pltpu.async_copy(
                h_hbm.at[idx_v.at[g * GROUP + j]],
                rows_v.at[pl.ds(j * CHUNK, CHUNK)],
                sem,
            )
            for j in range(GROUP)
        ]
        for d in descs:
            d.wait()
        base = pl.multiple_of((wid * CPT + g * GROUP) * CHUNK, CHUNK)
        pltpu.sync_copy(rows_v, out_hbm.at[pl.ds(base, GROUP * CHUNK)])


_gather = pl.kernel(
    _gather_body,
    out_type=jax.ShapeDtypeStruct((E_PAD, H), jnp.float32),
    mesh=_mesh,
    scratch_types=[
        pltpu.VMEM((CPT, CHUNK), jnp.int32),
        pltpu.VMEM((GROUP * CHUNK, H), jnp.float32),
        pltpu.SemaphoreType.DMA,
    ],
    compiler_params=_sc_params,
)


def _make_scatter(with_deg):
    def body(*refs):
        if with_deg:
            (m_hbm, dst2_hbm, zeros_hbm, zdeg_hbm, onecol_hbm,
             s_out, deg_out, idx_v, val_v, ones_v, sh_s, sh_deg) = refs
        else:
            (m_hbm, dst2_hbm, zeros_hbm,
             s_out, idx_v, val_v, sh_s) = refs
        cid = lax.axis_index("c")
        sid = lax.axis_index("s")
        wid = sid * NC + cid
        row0 = pl.multiple_of(sid * ROWS_PT, ROWS_PT)
        pltpu.sync_copy(zeros_hbm, sh_s.at[pl.ds(row0, ROWS_PT)])
        if with_deg:
            pltpu.sync_copy(zdeg_hbm, sh_deg.at[pl.ds(row0, ROWS_PT)])
            pltpu.sync_copy(onecol_hbm, ones_v)
        pltpu.sync_copy(dst2_hbm.at[pl.ds(wid * CPT, CPT)], idx_v)
        plsc.subcore_barrier()
        for g in range(CPT // GROUP):
            base = pl.multiple_of((wid * CPT + g * GROUP) * CHUNK, CHUNK)
            pltpu.sync_copy(m_hbm.at[pl.ds(base, GROUP * CHUNK)], val_v)
            for j in range(GROUP):
                idx_row = idx_v.at[g * GROUP + j]
                pltpu.sync_copy(
                    val_v.at[pl.ds(j * CHUNK, CHUNK)],
                    sh_s.at[idx_row],
                    add=True,
                )
                if with_deg:
                    pltpu.sync_copy(ones_v, sh_deg.at[idx_row], add=True)
        plsc.subcore_barrier()
        obase = pl.multiple_of(cid * N_PAD + row0, ROWS_PT)
        pltpu.sync_copy(sh_s.at[pl.ds(row0, ROWS_PT)],
                        s_out.at[pl.ds(obase, ROWS_PT)])
        if with_deg:
            pltpu.sync_copy(sh_deg.at[pl.ds(row0, ROWS_PT)],
                            deg_out.at[pl.ds(obase, ROWS_PT)])

    out_type = [jax.ShapeDtypeStruct((NC * N_PAD, H), jnp.float32)]
    scratch = [
        pltpu.VMEM((CPT, CHUNK), jnp.int32),
        pltpu.VMEM((GROUP * CHUNK, H), jnp.float32),
    ]
    if with_deg:
        out_type.append(jax.ShapeDtypeStruct((NC * N_PAD, DEGW), jnp.float32))
        scratch.append(pltpu.VMEM((CHUNK, DEGW), jnp.float32))
    scratch.append(pltpu.VMEM_SHARED((N_PAD, H), jnp.float32))
    if with_deg:
        scratch.append(pltpu.VMEM_SHARED((N_PAD, DEGW), jnp.float32))
    return pl.kernel(
        body,
        out_type=tuple(out_type) if with_deg else out_type[0],
        mesh=_mesh,
        scratch_types=scratch,
        compiler_params=_sc_params,
    )


_scatter_deg = _make_scatter(True)
_scatter = _make_scatter(False)


def _proj_body(nf_ref, wp_ref, bp_ref, out_ref):
    out_ref[...] = (
        jnp.dot(nf_ref[...], wp_ref[...], preferred_element_type=jnp.float32)
        + bp_ref[...]
    )


_NB = 1000

_proj = pl.pallas_call(
    _proj_body,
    grid=(N // _NB,),
    in_specs=[
        pl.BlockSpec((_NB, F_IN), lambda i: (i, 0)),
        pl.BlockSpec((F_IN, H), lambda i: (0, 0)),
        pl.BlockSpec((1, H), lambda i: (0, 0)),
    ],
    out_specs=pl.BlockSpec((_NB, H), lambda i: (i, 0)),
    out_shape=jax.ShapeDtypeStruct((N, H), jnp.float32),
)


_BE = 512


_BL = _BE // 4


def _msg_body(efT_ref, hs4_ref, weT_ref, be_ref, out_ref):
    z = (
        jnp.dot(weT_ref[...], efT_ref[...], preferred_element_type=jnp.float32)
        + be_ref[...]
    )
    z = jnp.maximum(z, 0.0)
    T = hs4_ref[...].T
    accs = []
    for q in range(4):
        zq = z[:, _BL * q:_BL * (q + 1)]
        hq = T[H * q:H * q + H, :]
        acc = zq[0:H, :] * hq[0:1, :]
        for i in range(1, H):
            acc = acc + zq[H * i:H * i + H, :] * hq[i:i + 1, :]
        accs.append(acc)
    out_ref[...] = jnp.concatenate(accs, axis=0).T


_msg = pl.pallas_call(
    _msg_body,
    grid=(E_PAD // _BE,),
    in_specs=[
        pl.BlockSpec((F_E, _BE), lambda i: (0, i)),
        pl.BlockSpec((_BL, 4 * H), lambda i: (i, 0)),
        pl.BlockSpec((H * H, F_E), lambda i: (0, 0)),
        pl.BlockSpec((H * H, 1), lambda i: (0, 0)),
    ],
    out_specs=pl.BlockSpec((_BL, 4 * H), lambda i: (i, 0)),
    out_shape=jax.ShapeDtypeStruct((E_PAD // 4, 4 * H), jnp.float32),
)


def _combine_body(sp_ref, dp_ref, b_ref, out_ref):
    s = sp_ref[0] + sp_ref[1]
    deg = jnp.maximum(dp_ref[0, :, 0:1] + dp_ref[1, :, 0:1], 1.0)
    out_ref[...] = jnp.maximum(s / deg + b_ref[...], 0.0)


_combine = pl.pallas_call(
    _combine_body,
    grid=(N // _NB,),
    in_specs=[
        pl.BlockSpec((NC, _NB, H), lambda i: (0, i, 0)),
        pl.BlockSpec((NC, _NB, DEGW), lambda i: (0, i, 0)),
        pl.BlockSpec((1, H), lambda i: (0, 0)),
    ],
    out_specs=pl.BlockSpec((_NB, H), lambda i: (i, 0)),
    out_shape=jax.ShapeDtypeStruct((N, H), jnp.float32),
)


def kernel(node_features, edge_index, edge_features, W_proj, b_proj,
           W_e, b_e, bias1, bias2):
    src = edge_index[0]
    dst = edge_index[1]
    src2 = jnp.concatenate(
        [src, jnp.zeros((E_PAD - E,), jnp.int32)]
    ).reshape(E_PAD // CHUNK, CHUNK)
    dst2 = jnp.concatenate(
        [dst, jnp.full((E_PAD - E,), N_PAD - 1, jnp.int32)]
    ).reshape(E_PAD // CHUNK, CHUNK)
    efT = (
        jnp.concatenate(
            [edge_features, jnp.zeros((E_PAD - E, F_E), jnp.float32)], axis=0
        )
        .reshape(E_PAD // _BE, _BL, 4, F_E)
        .transpose(0, 2, 1, 3)
        .reshape(E_PAD, F_E)
        .T
    )
    weT = W_e.T
    zeros = jnp.zeros((ROWS_PT, H), jnp.float32)
    zdeg = jnp.zeros((ROWS_PT, DEGW), jnp.float32)
    onecol = jnp.zeros((CHUNK, DEGW), jnp.float32).at[:, 0].set(1.0)
    be2 = b_e.reshape(H * H, 1)

    h = _proj(node_features, W_proj, b_proj.reshape(1, H))

    hs = _gather(h, src2)
    m = _msg(efT, hs.reshape(E_PAD // 4, 4 * H), weT, be2).reshape(E_PAD, H)
    s_flat, deg_flat = _scatter_deg(m, dst2, zeros, zdeg, onecol)
    sp = s_flat.reshape(NC, N_PAD, H)
    dp = deg_flat.reshape(NC, N_PAD, DEGW)
    h = _combine(sp, dp, bias1.reshape(1, H))

    hs = _gather(h, src2)
    m = _msg(efT, hs.reshape(E_PAD // 4, 4 * H), weT, be2).reshape(E_PAD, H)
    s_flat = _scatter(m, dst2, zeros)
    sp = s_flat.reshape(NC, N_PAD, H)
    h = _combine(sp, dp, bias2.reshape(1, H))
    return h


# speedup vs baseline: 4.0980x; 1.1202x over previous
import jax
import jax.numpy as jnp
from jax import lax
from jax.experimental import pallas as pl
from jax.experimental.pallas import tpu as pltpu
from jax.experimental.pallas import tpu_sc as plsc

N = 10000
E = 160000
F_IN = 128
F_E = 16
H = 32

NC = 2
NS = 16
NW = NC * NS
CHUNK = 128
GROUP = 10
N_PAD = 10240
E_PAD = NW * 40 * CHUNK
CPT = E_PAD // NW // CHUNK
ROWS_PT = N_PAD // NS
DEGW = 16

_mesh = plsc.VectorSubcoreMesh(core_axis_name="c", subcore_axis_name="s")
_sc_params = pltpu.CompilerParams(use_tc_tiling_on_sc=False)


_HPT = N // NS


def _gather_body(h_hbm, src2_hbm, out_hbm, idx_v, rows_v0, rows_v1,
                 sh_h, sem, sem_wb):
    cid = lax.axis_index("c")
    sid = lax.axis_index("s")
    wid = sid * NC + cid
    pltpu.sync_copy(h_hbm.at[pl.ds(sid * _HPT, _HPT)],
                    sh_h.at[pl.ds(sid * _HPT, _HPT)])
    pltpu.sync_copy(src2_hbm.at[pl.ds(wid * CPT, CPT)], idx_v)
    plsc.subcore_barrier()
    bufs = (rows_v0, rows_v1)
    wb = [None, None]
    for g in range(CPT // GROUP):
        buf = bufs[g % 2]
        if wb[g % 2] is not None:
            wb[g % 2].wait()
        descs = [
            pltpu.async_copy(
                sh_h.at[idx_v.at[g * GROUP + j]],
                buf.at[pl.ds(j * CHUNK, CHUNK)],
                sem,
            )
            for j in range(GROUP)
        ]
        for d in descs:
            d.wait()
        base = pl.multiple_of((wid * CPT + g * GROUP) * CHUNK, CHUNK)
        wb[g % 2] = pltpu.async_copy(
            buf, out_hbm.at[pl.ds(base, GROUP * CHUNK)], sem_wb)
    wb[0].wait()
    wb[1].wait()


_gather = pl.kernel(
    _gather_body,
    out_type=jax.ShapeDtypeStruct((E_PAD, H), jnp.float32),
    mesh=_mesh,
    scratch_types=[
        pltpu.VMEM((CPT, CHUNK), jnp.int32),
        pltpu.VMEM((GROUP * CHUNK, H), jnp.float32),
        pltpu.VMEM((GROUP * CHUNK, H), jnp.float32),
        pltpu.VMEM_SHARED((N, H), jnp.float32),
        pltpu.SemaphoreType.DMA,
        pltpu.SemaphoreType.DMA,
    ],
    compiler_params=_sc_params,
)


def _make_scatter(with_deg):
    def body(*refs):
        if with_deg:
            (m_hbm, dst2_hbm, zeros_hbm, zdeg_hbm, onecol_hbm,
             s_out, deg_out, idx_v, val_v, ones_v, sh_s, sh_deg) = refs
        else:
            (m_hbm, dst2_hbm, zeros_hbm,
             s_out, idx_v, val_v, sh_s) = refs
        cid = lax.axis_index("c")
        sid = lax.axis_index("s")
        wid = sid * NC + cid
        row0 = pl.multiple_of(sid * ROWS_PT, ROWS_PT)
        pltpu.sync_copy(zeros_hbm, sh_s.at[pl.ds(row0, ROWS_PT)])
        if with_deg:
            pltpu.sync_copy(zdeg_hbm, sh_deg.at[pl.ds(row0, ROWS_PT)])
            pltpu.sync_copy(onecol_hbm, ones_v)
        pltpu.sync_copy(dst2_hbm.at[pl.ds(wid * CPT, CPT)], idx_v)
        plsc.subcore_barrier()
        for g in range(CPT // GROUP):
            base = pl.multiple_of((wid * CPT + g * GROUP) * CHUNK, CHUNK)
            pltpu.sync_copy(m_hbm.at[pl.ds(base, GROUP * CHUNK)], val_v)
            for j in range(GROUP):
                idx_row = idx_v.at[g * GROUP + j]
                pltpu.sync_copy(
                    val_v.at[pl.ds(j * CHUNK, CHUNK)],
                    sh_s.at[idx_row],
                    add=True,
                )
                if with_deg:
                    pltpu.sync_copy(ones_v, sh_deg.at[idx_row], add=True)
        plsc.subcore_barrier()
        obase = pl.multiple_of(cid * N_PAD + row0, ROWS_PT)
        pltpu.sync_copy(sh_s.at[pl.ds(row0, ROWS_PT)],
                        s_out.at[pl.ds(obase, ROWS_PT)])
        if with_deg:
            pltpu.sync_copy(sh_deg.at[pl.ds(row0, ROWS_PT)],
                            deg_out.at[pl.ds(obase, ROWS_PT)])

    out_type = [jax.ShapeDtypeStruct((NC * N_PAD, H), jnp.float32)]
    scratch = [
        pltpu.VMEM((CPT, CHUNK), jnp.int32),
        pltpu.VMEM((GROUP * CHUNK, H), jnp.float32),
    ]
    if with_deg:
        out_type.append(jax.ShapeDtypeStruct((NC * N_PAD, DEGW), jnp.float32))
        scratch.append(pltpu.VMEM((CHUNK, DEGW), jnp.float32))
    scratch.append(pltpu.VMEM_SHARED((N_PAD, H), jnp.float32))
    if with_deg:
        scratch.append(pltpu.VMEM_SHARED((N_PAD, DEGW), jnp.float32))
    return pl.kernel(
        body,
        out_type=tuple(out_type) if with_deg else out_type[0],
        mesh=_mesh,
        scratch_types=scratch,
        compiler_params=_sc_params,
    )


_scatter_deg = _make_scatter(True)
_scatter = _make_scatter(False)


def _proj_body(nf_ref, wp_ref, bp_ref, out_ref):
    out_ref[...] = (
        jnp.dot(nf_ref[...], wp_ref[...], preferred_element_type=jnp.float32)
        + bp_ref[...]
    )


_NB = 1000

_proj = pl.pallas_call(
    _proj_body,
    grid=(N // _NB,),
    in_specs=[
        pl.BlockSpec((_NB, F_IN), lambda i: (i, 0)),
        pl.BlockSpec((F_IN, H), lambda i: (0, 0)),
        pl.BlockSpec((1, H), lambda i: (0, 0)),
    ],
    out_specs=pl.BlockSpec((_NB, H), lambda i: (i, 0)),
    out_shape=jax.ShapeDtypeStruct((N, H), jnp.float32),
)


_BE = 512


_BL = _BE // 4


def _msg_body(efT_ref, hs4_ref, weT_ref, be_ref, out_ref):
    z = (
        jnp.dot(weT_ref[...], efT_ref[...], preferred_element_type=jnp.float32)
        + be_ref[...]
    )
    z = jnp.maximum(z, 0.0)
    T = hs4_ref[...].T
    accs = []
    for q in range(4):
        zq = z[:, _BL * q:_BL * (q + 1)]
        hq = T[H * q:H * q + H, :]
        acc = zq[0:H, :] * hq[0:1, :]
        for i in range(1, H):
            acc = acc + zq[H * i:H * i + H, :] * hq[i:i + 1, :]
        accs.append(acc)
    out_ref[...] = jnp.concatenate(accs, axis=0).T


_msg = pl.pallas_call(
    _msg_body,
    grid=(E_PAD // _BE,),
    in_specs=[
        pl.BlockSpec((F_E, _BE), lambda i: (0, i)),
        pl.BlockSpec((_BL, 4 * H), lambda i: (i, 0)),
        pl.BlockSpec((H * H, F_E), lambda i: (0, 0)),
        pl.BlockSpec((H * H, 1), lambda i: (0, 0)),
    ],
    out_specs=pl.BlockSpec((_BL, 4 * H), lambda i: (i, 0)),
    out_shape=jax.ShapeDtypeStruct((E_PAD // 4, 4 * H), jnp.float32),
)


def _combine_body(sp_ref, dp_ref, b_ref, out_ref):
    s = sp_ref[0] + sp_ref[1]
    deg = jnp.maximum(dp_ref[0, :, 0:1] + dp_ref[1, :, 0:1], 1.0)
    out_ref[...] = jnp.maximum(s / deg + b_ref[...], 0.0)


_combine = pl.pallas_call(
    _combine_body,
    grid=(N // _NB,),
    in_specs=[
        pl.BlockSpec((NC, _NB, H), lambda i: (0, i, 0)),
        pl.BlockSpec((NC, _NB, DEGW), lambda i: (0, i, 0)),
        pl.BlockSpec((1, H), lambda i: (0, 0)),
    ],
    out_specs=pl.BlockSpec((_NB, H), lambda i: (i, 0)),
    out_shape=jax.ShapeDtypeStruct((N, H), jnp.float32),
)


def kernel(node_features, edge_index, edge_features, W_proj, b_proj,
           W_e, b_e, bias1, bias2):
    src = edge_index[0]
    dst = edge_index[1]
    src2 = jnp.concatenate(
        [src, jnp.zeros((E_PAD - E,), jnp.int32)]
    ).reshape(E_PAD // CHUNK, CHUNK)
    dst2 = jnp.concatenate(
        [dst, jnp.full((E_PAD - E,), N_PAD - 1, jnp.int32)]
    ).reshape(E_PAD // CHUNK, CHUNK)
    efT = (
        jnp.concatenate(
            [edge_features, jnp.zeros((E_PAD - E, F_E), jnp.float32)], axis=0
        )
        .astype(jnp.bfloat16)
        .reshape(E_PAD // _BE, _BL, 4, F_E)
        .transpose(0, 2, 1, 3)
        .reshape(E_PAD, F_E)
        .T
    )
    weT = W_e.T.astype(jnp.bfloat16)
    zeros = jnp.zeros((ROWS_PT, H), jnp.float32)
    zdeg = jnp.zeros((ROWS_PT, DEGW), jnp.float32)
    onecol = jnp.zeros((CHUNK, DEGW), jnp.float32).at[:, 0].set(1.0)
    be2 = b_e.reshape(H * H, 1)

    h = _proj(node_features, W_proj, b_proj.reshape(1, H))

    hs = _gather(h, src2)
    m = _msg(efT, hs.reshape(E_PAD // 4, 4 * H), weT, be2).reshape(E_PAD, H)
    s_flat, deg_flat = _scatter_deg(m, dst2, zeros, zdeg, onecol)
    sp = s_flat.reshape(NC, N_PAD, H)
    dp = deg_flat.reshape(NC, N_PAD, DEGW)
    h = _combine(sp, dp, bias1.reshape(1, H))

    hs = _gather(h, src2)
    m = _msg(efT, hs.reshape(E_PAD // 4, 4 * H), weT, be2).reshape(E_PAD, H)
    s_flat = _scatter(m, dst2, zeros)
    sp = s_flat.reshape(NC, N_PAD, H)
    h = _combine(sp, dp, bias2.reshape(1, H))
    return h


# speedup vs baseline: 5.2546x; 1.2823x over previous
import jax
import jax.numpy as jnp
from jax import lax
from jax.experimental import pallas as pl
from jax.experimental.pallas import tpu as pltpu
from jax.experimental.pallas import tpu_sc as plsc

N = 10000
E = 160000
F_IN = 128
F_E = 16
H = 32

NC = 2
NS = 16
NW = NC * NS
CHUNK = 128
GROUP = 10
N_PAD = 10240
E_PAD = NW * 40 * CHUNK
CPT = E_PAD // NW // CHUNK
ROWS_PT = N_PAD // NS
DEGW = 16

_mesh = plsc.VectorSubcoreMesh(core_axis_name="c", subcore_axis_name="s")
_sc_params = pltpu.CompilerParams(use_tc_tiling_on_sc=False)


_HPT = N // NS


def _gather_body(h_hbm, src2_hbm, out_hbm, idx_v, rows_v0, rows_v1,
                 sh_h, sem, sem_wb):
    cid = lax.axis_index("c")
    sid = lax.axis_index("s")
    wid = sid * NC + cid
    pltpu.sync_copy(h_hbm.at[pl.ds(sid * _HPT, _HPT)],
                    sh_h.at[pl.ds(sid * _HPT, _HPT)])
    pltpu.sync_copy(src2_hbm.at[pl.ds(wid * CPT, CPT)], idx_v)
    plsc.subcore_barrier()
    bufs = (rows_v0, rows_v1)
    wb = [None, None]
    for g in range(CPT // GROUP):
        buf = bufs[g % 2]
        if wb[g % 2] is not None:
            wb[g % 2].wait()
        descs = [
            pltpu.async_copy(
                sh_h.at[idx_v.at[g * GROUP + j]],
                buf.at[pl.ds(j * CHUNK, CHUNK)],
                sem,
            )
            for j in range(GROUP)
        ]
        for d in descs:
            d.wait()
        base = pl.multiple_of((wid * CPT + g * GROUP) * CHUNK, CHUNK)
        wb[g % 2] = pltpu.async_copy(
            buf, out_hbm.at[pl.ds(base, GROUP * CHUNK)], sem_wb)
    wb[0].wait()
    wb[1].wait()


_gather = pl.kernel(
    _gather_body,
    out_type=jax.ShapeDtypeStruct((E_PAD, H), jnp.float32),
    mesh=_mesh,
    scratch_types=[
        pltpu.VMEM((CPT, CHUNK), jnp.int32),
        pltpu.VMEM((GROUP * CHUNK, H), jnp.float32),
        pltpu.VMEM((GROUP * CHUNK, H), jnp.float32),
        pltpu.VMEM_SHARED((N, H), jnp.float32),
        pltpu.SemaphoreType.DMA,
        pltpu.SemaphoreType.DMA,
    ],
    compiler_params=_sc_params,
)


def _make_scatter(with_deg):
    def body(*refs):
        if with_deg:
            (m_hbm, dst2_hbm, zeros_hbm, zdeg_hbm, onecol_hbm,
             s_out, deg_out, idx_v, val_v, ones_v, sh_s, sh_deg) = refs
        else:
            (m_hbm, dst2_hbm, zeros_hbm,
             s_out, idx_v, val_v, sh_s) = refs
        cid = lax.axis_index("c")
        sid = lax.axis_index("s")
        wid = sid * NC + cid
        row0 = pl.multiple_of(sid * ROWS_PT, ROWS_PT)
        pltpu.sync_copy(zeros_hbm, sh_s.at[pl.ds(row0, ROWS_PT)])
        if with_deg:
            pltpu.sync_copy(zdeg_hbm, sh_deg.at[pl.ds(row0, ROWS_PT)])
            pltpu.sync_copy(onecol_hbm, ones_v)
        pltpu.sync_copy(dst2_hbm.at[pl.ds(wid * CPT, CPT)], idx_v)
        plsc.subcore_barrier()
        for g in range(CPT // GROUP):
            base = pl.multiple_of((wid * CPT + g * GROUP) * CHUNK, CHUNK)
            pltpu.sync_copy(m_hbm.at[pl.ds(base, GROUP * CHUNK)], val_v)
            for j in range(GROUP):
                idx_row = idx_v.at[g * GROUP + j]
                pltpu.sync_copy(
                    val_v.at[pl.ds(j * CHUNK, CHUNK)],
                    sh_s.at[idx_row],
                    add=True,
                )
                if with_deg:
                    pltpu.sync_copy(ones_v, sh_deg.at[idx_row], add=True)
        plsc.subcore_barrier()
        obase = pl.multiple_of(cid * N_PAD + row0, ROWS_PT)
        pltpu.sync_copy(sh_s.at[pl.ds(row0, ROWS_PT)],
                        s_out.at[pl.ds(obase, ROWS_PT)])
        if with_deg:
            pltpu.sync_copy(sh_deg.at[pl.ds(row0, ROWS_PT)],
                            deg_out.at[pl.ds(obase, ROWS_PT)])

    out_type = [jax.ShapeDtypeStruct((NC * N_PAD, H), jnp.float32)]
    scratch = [
        pltpu.VMEM((CPT, CHUNK), jnp.int32),
        pltpu.VMEM((GROUP * CHUNK, H), jnp.float32),
    ]
    if with_deg:
        out_type.append(jax.ShapeDtypeStruct((NC * N_PAD, DEGW), jnp.float32))
        scratch.append(pltpu.VMEM((CHUNK, DEGW), jnp.float32))
    scratch.append(pltpu.VMEM_SHARED((N_PAD, H), jnp.float32))
    if with_deg:
        scratch.append(pltpu.VMEM_SHARED((N_PAD, DEGW), jnp.float32))
    return pl.kernel(
        body,
        out_type=tuple(out_type) if with_deg else out_type[0],
        mesh=_mesh,
        scratch_types=scratch,
        compiler_params=_sc_params,
    )


_scatter_deg = _make_scatter(True)
_scatter = _make_scatter(False)


def _proj_body(nf_ref, wp_ref, bp_ref, out_ref):
    out_ref[...] = (
        jnp.dot(nf_ref[...], wp_ref[...], preferred_element_type=jnp.float32)
        + bp_ref[...]
    )


_NB = 1000

_proj = pl.pallas_call(
    _proj_body,
    grid=(N // _NB,),
    in_specs=[
        pl.BlockSpec((_NB, F_IN), lambda i: (i, 0)),
        pl.BlockSpec((F_IN, H), lambda i: (0, 0)),
        pl.BlockSpec((1, H), lambda i: (0, 0)),
    ],
    out_specs=pl.BlockSpec((_NB, H), lambda i: (i, 0)),
    out_shape=jax.ShapeDtypeStruct((N, H), jnp.float32),
)


_BE = 1024


_BL = _BE // 4


def _msg_body(ef_ref, hs4_ref, weT_ref, out_ref):
    eT = ef_ref[...].T
    ones = jnp.ones((1, _BE), jnp.bfloat16)
    z = jnp.dot(
        weT_ref[...],
        jnp.concatenate([eT, ones], axis=0),
        preferred_element_type=jnp.float32,
    )
    z = jnp.maximum(z, 0.0)
    T = hs4_ref[...].T
    accs = []
    for q in range(4):
        zq = z[:, _BL * q:_BL * (q + 1)]
        hq = T[H * q:H * q + H, :]
        acc = zq[0:H, :] * hq[0:1, :]
        for i in range(1, H):
            acc = acc + zq[H * i:H * i + H, :] * hq[i:i + 1, :]
        accs.append(acc)
    out_ref[...] = jnp.concatenate(accs, axis=0).T


_msg = pl.pallas_call(
    _msg_body,
    grid=(E_PAD // _BE,),
    in_specs=[
        pl.BlockSpec((_BE, F_E), lambda i: (i, 0)),
        pl.BlockSpec((_BL, 4 * H), lambda i: (i, 0)),
        pl.BlockSpec((H * H, F_E + 1), lambda i: (0, 0)),
    ],
    out_specs=pl.BlockSpec((_BL, 4 * H), lambda i: (i, 0)),
    out_shape=jax.ShapeDtypeStruct((E_PAD // 4, 4 * H), jnp.float32),
)


def _combine_body(sp_ref, dp_ref, b_ref, out_ref):
    s = sp_ref[0] + sp_ref[1]
    deg = jnp.maximum(dp_ref[0, :, 0:1] + dp_ref[1, :, 0:1], 1.0)
    out_ref[...] = jnp.maximum(s / deg + b_ref[...], 0.0)


_combine = pl.pallas_call(
    _combine_body,
    grid=(N // _NB,),
    in_specs=[
        pl.BlockSpec((NC, _NB, H), lambda i: (0, i, 0)),
        pl.BlockSpec((NC, _NB, DEGW), lambda i: (0, i, 0)),
        pl.BlockSpec((1, H), lambda i: (0, 0)),
    ],
    out_specs=pl.BlockSpec((_NB, H), lambda i: (i, 0)),
    out_shape=jax.ShapeDtypeStruct((N, H), jnp.float32),
)


def kernel(node_features, edge_index, edge_features, W_proj, b_proj,
           W_e, b_e, bias1, bias2):
    src = edge_index[0]
    dst = edge_index[1]

    def _perm(a, fill):
        return (
            jnp.concatenate([a, jnp.full((E_PAD - E,), fill, jnp.int32)])
            .reshape(E_PAD // _BE, 4, _BL)
            .transpose(0, 2, 1)
            .reshape(E_PAD // CHUNK, CHUNK)
        )

    src2 = _perm(src, 0)
    dst2 = _perm(dst, N_PAD - 1)
    ef_p = jnp.concatenate(
        [edge_features, jnp.zeros((E_PAD - E, F_E), jnp.float32)], axis=0
    ).astype(jnp.bfloat16)
    weT = jnp.concatenate(
        [W_e.T, b_e[:, None]], axis=1
    ).astype(jnp.bfloat16)
    zeros = jnp.zeros((ROWS_PT, H), jnp.float32)
    zdeg = jnp.zeros((ROWS_PT, DEGW), jnp.float32)
    onecol = jnp.zeros((CHUNK, DEGW), jnp.float32).at[:, 0].set(1.0)
    h = _proj(node_features, W_proj, b_proj.reshape(1, H))

    hs = _gather(h, src2)
    m = _msg(ef_p, hs.reshape(E_PAD // 4, 4 * H), weT).reshape(E_PAD, H)
    s_flat, deg_flat = _scatter_deg(m, dst2, zeros, zdeg, onecol)
    sp = s_flat.reshape(NC, N_PAD, H)
    dp = deg_flat.reshape(NC, N_PAD, DEGW)
    h = _combine(sp, dp, bias1.reshape(1, H))

    hs = _gather(h, src2)
    m = _msg(ef_p, hs.reshape(E_PAD // 4, 4 * H), weT).reshape(E_PAD, H)
    s_flat = _scatter(m, dst2, zeros)
    sp = s_flat.reshape(NC, N_PAD, H)
    h = _combine(sp, dp, bias2.reshape(1, H))
    return h


# speedup vs baseline: 5.7320x; 1.0908x over previous
import jax
import jax.numpy as jnp
from jax import lax
from jax.experimental import pallas as pl
from jax.experimental.pallas import tpu as pltpu
from jax.experimental.pallas import tpu_sc as plsc

N = 10000
E = 160000
F_IN = 128
F_E = 16
H = 32

NC = 2
NS = 16
NW = NC * NS
CHUNK = 128
GROUP = 10
N_PAD = 10240
E_PAD = NW * 40 * CHUNK
CPT = E_PAD // NW // CHUNK
ROWS_PT = N_PAD // NS
DEGW = 16

_mesh = plsc.VectorSubcoreMesh(core_axis_name="c", subcore_axis_name="s")
_sc_params = pltpu.CompilerParams(use_tc_tiling_on_sc=False)


_HPT = N // NS


def _gather_body(h_hbm, src2_hbm, out_hbm, idx_v, rows_v0, rows_v1,
                 sh_h, sem, sem_wb):
    cid = lax.axis_index("c")
    sid = lax.axis_index("s")
    wid = sid * NC + cid
    pltpu.sync_copy(h_hbm.at[pl.ds(sid * _HPT, _HPT)],
                    sh_h.at[pl.ds(sid * _HPT, _HPT)])
    pltpu.sync_copy(src2_hbm.at[pl.ds(wid * CPT, CPT)], idx_v)
    plsc.subcore_barrier()
    bufs = (rows_v0, rows_v1)
    wb = [None, None]
    for g in range(CPT // GROUP):
        buf = bufs[g % 2]
        if wb[g % 2] is not None:
            wb[g % 2].wait()
        descs = [
            pltpu.async_copy(
                sh_h.at[idx_v.at[g * GROUP + j]],
                buf.at[pl.ds(j * CHUNK, CHUNK)],
                sem,
            )
            for j in range(GROUP)
        ]
        for d in descs:
            d.wait()
        base = pl.multiple_of((wid * CPT + g * GROUP) * CHUNK, CHUNK)
        wb[g % 2] = pltpu.async_copy(
            buf, out_hbm.at[pl.ds(base, GROUP * CHUNK)], sem_wb)
    wb[0].wait()
    wb[1].wait()


_gather = pl.kernel(
    _gather_body,
    out_type=jax.ShapeDtypeStruct((E_PAD, H), jnp.float32),
    mesh=_mesh,
    scratch_types=[
        pltpu.VMEM((CPT, CHUNK), jnp.int32),
        pltpu.VMEM((GROUP * CHUNK, H), jnp.float32),
        pltpu.VMEM((GROUP * CHUNK, H), jnp.float32),
        pltpu.VMEM_SHARED((N, H), jnp.float32),
        pltpu.SemaphoreType.DMA,
        pltpu.SemaphoreType.DMA,
    ],
    compiler_params=_sc_params,
)


def _make_scatter(with_deg):
    def body(*refs):
        if with_deg:
            (m_hbm, dst2_hbm, zeros_hbm, zdeg_hbm, onecol_hbm,
             s_out, deg_out, idx_v, val_v, ones_v, sh_s, sh_deg) = refs
        else:
            (m_hbm, dst2_hbm, zeros_hbm,
             s_out, idx_v, val_v, sh_s) = refs
        cid = lax.axis_index("c")
        sid = lax.axis_index("s")
        wid = sid * NC + cid
        row0 = pl.multiple_of(sid * ROWS_PT, ROWS_PT)
        pltpu.sync_copy(zeros_hbm, sh_s.at[pl.ds(row0, ROWS_PT)])
        if with_deg:
            pltpu.sync_copy(zdeg_hbm, sh_deg.at[pl.ds(row0, ROWS_PT)])
            pltpu.sync_copy(onecol_hbm, ones_v)
        pltpu.sync_copy(dst2_hbm.at[pl.ds(wid * CPT, CPT)], idx_v)
        plsc.subcore_barrier()
        for g in range(CPT // GROUP):
            base = pl.multiple_of((wid * CPT + g * GROUP) * CHUNK, CHUNK)
            pltpu.sync_copy(m_hbm.at[pl.ds(base, GROUP * CHUNK)], val_v)
            for j in range(GROUP):
                idx_row = idx_v.at[g * GROUP + j]
                pltpu.sync_copy(
                    val_v.at[pl.ds(j * CHUNK, CHUNK)],
                    sh_s.at[idx_row],
                    add=True,
                )
                if with_deg:
                    pltpu.sync_copy(ones_v, sh_deg.at[idx_row], add=True)
        plsc.subcore_barrier()
        obase = pl.multiple_of(cid * N_PAD + row0, ROWS_PT)
        pltpu.sync_copy(sh_s.at[pl.ds(row0, ROWS_PT)],
                        s_out.at[pl.ds(obase, ROWS_PT)])
        if with_deg:
            pltpu.sync_copy(sh_deg.at[pl.ds(row0, ROWS_PT)],
                            deg_out.at[pl.ds(obase, ROWS_PT)])

    out_type = [jax.ShapeDtypeStruct((NC * N_PAD, H), jnp.float32)]
    scratch = [
        pltpu.VMEM((CPT, CHUNK), jnp.int32),
        pltpu.VMEM((GROUP * CHUNK, H), jnp.float32),
    ]
    if with_deg:
        out_type.append(jax.ShapeDtypeStruct((NC * N_PAD, DEGW), jnp.float32))
        scratch.append(pltpu.VMEM((CHUNK, DEGW), jnp.float32))
    scratch.append(pltpu.VMEM_SHARED((N_PAD, H), jnp.float32))
    if with_deg:
        scratch.append(pltpu.VMEM_SHARED((N_PAD, DEGW), jnp.float32))
    return pl.kernel(
        body,
        out_type=tuple(out_type) if with_deg else out_type[0],
        mesh=_mesh,
        scratch_types=scratch,
        compiler_params=_sc_params,
    )


_scatter_deg = _make_scatter(True)
_scatter = _make_scatter(False)


def _proj_body(nf_ref, wp_ref, bp_ref, out_ref):
    out_ref[...] = (
        jnp.dot(nf_ref[...], wp_ref[...], preferred_element_type=jnp.float32)
        + bp_ref[...]
    )


_NB = 1000

_proj = pl.pallas_call(
    _proj_body,
    grid=(N // _NB,),
    in_specs=[
        pl.BlockSpec((_NB, F_IN), lambda i: (i, 0)),
        pl.BlockSpec((F_IN, H), lambda i: (0, 0)),
        pl.BlockSpec((1, H), lambda i: (0, 0)),
    ],
    out_specs=pl.BlockSpec((_NB, H), lambda i: (i, 0)),
    out_shape=jax.ShapeDtypeStruct((N, H), jnp.float32),
)


_BE = 2048


_BL = _BE // 4


def _msg_body(ef_ref, hs4_ref, weT_ref, out_ref):
    eT = ef_ref[...].T
    ones = jnp.ones((1, _BE), jnp.bfloat16)
    z = jnp.dot(
        weT_ref[...],
        jnp.concatenate([eT, ones], axis=0),
        preferred_element_type=jnp.float32,
    )
    z = jnp.maximum(z, 0.0)
    T = hs4_ref[...].T
    accs = []
    for q in range(4):
        zq = z[:, _BL * q:_BL * (q + 1)]
        hq = T[H * q:H * q + H, :]
        acc = zq[0:H, :] * hq[0:1, :]
        for i in range(1, H):
            acc = acc + zq[H * i:H * i + H, :] * hq[i:i + 1, :]
        accs.append(acc)
    out_ref[...] = jnp.concatenate(accs, axis=0).T


_msg = pl.pallas_call(
    _msg_body,
    grid=(E_PAD // _BE,),
    in_specs=[
        pl.BlockSpec((_BE, F_E), lambda i: (i, 0)),
        pl.BlockSpec((_BL, 4 * H), lambda i: (i, 0)),
        pl.BlockSpec((H * H, F_E + 1), lambda i: (0, 0)),
    ],
    out_specs=pl.BlockSpec((_BL, 4 * H), lambda i: (i, 0)),
    out_shape=jax.ShapeDtypeStruct((E_PAD // 4, 4 * H), jnp.float32),
)


def _combine_body(sp_ref, dp_ref, b_ref, out_ref):
    s = sp_ref[0] + sp_ref[1]
    deg = jnp.maximum(dp_ref[0, :, 0:1] + dp_ref[1, :, 0:1], 1.0)
    out_ref[...] = jnp.maximum(s / deg + b_ref[...], 0.0)


_combine = pl.pallas_call(
    _combine_body,
    grid=(N // _NB,),
    in_specs=[
        pl.BlockSpec((NC, _NB, H), lambda i: (0, i, 0)),
        pl.BlockSpec((NC, _NB, DEGW), lambda i: (0, i, 0)),
        pl.BlockSpec((1, H), lambda i: (0, 0)),
    ],
    out_specs=pl.BlockSpec((_NB, H), lambda i: (i, 0)),
    out_shape=jax.ShapeDtypeStruct((N, H), jnp.float32),
)


def kernel(node_features, edge_index, edge_features, W_proj, b_proj,
           W_e, b_e, bias1, bias2):
    pad_cols = jnp.concatenate(
        [jnp.zeros((1, E_PAD - E), jnp.int32),
         jnp.full((1, E_PAD - E), N_PAD - 1, jnp.int32)], axis=0
    )
    ei2 = (
        jnp.concatenate([edge_index, pad_cols], axis=1)
        .reshape(2, E_PAD // _BE, 4, _BL)
        .transpose(0, 1, 3, 2)
        .reshape(2, E_PAD // CHUNK, CHUNK)
    )
    src2 = ei2[0]
    dst2 = ei2[1]
    ef_p = jnp.pad(
        edge_features, ((0, E_PAD - E), (0, 0))
    ).astype(jnp.bfloat16)
    weT = jnp.concatenate(
        [W_e.T, b_e[:, None]], axis=1
    ).astype(jnp.bfloat16)
    zeros = jnp.zeros((ROWS_PT, H), jnp.float32)
    zdeg = jnp.zeros((ROWS_PT, DEGW), jnp.float32)
    onecol = jnp.zeros((CHUNK, DEGW), jnp.float32).at[:, 0].set(1.0)
    h = _proj(node_features, W_proj, b_proj.reshape(1, H))

    hs = _gather(h, src2)
    m = _msg(ef_p, hs.reshape(E_PAD // 4, 4 * H), weT).reshape(E_PAD, H)
    s_flat, deg_flat = _scatter_deg(m, dst2, zeros, zdeg, onecol)
    sp = s_flat.reshape(NC, N_PAD, H)
    dp = deg_flat.reshape(NC, N_PAD, DEGW)
    h = _combine(sp, dp, bias1.reshape(1, H))

    hs = _gather(h, src2)
    m = _msg(ef_p, hs.reshape(E_PAD // 4, 4 * H), weT).reshape(E_PAD, H)
    s_flat = _scatter(m, dst2, zeros)
    sp = s_flat.reshape(NC, N_PAD, H)
    h = _combine(sp, dp, bias2.reshape(1, H))
    return h
